# Initial kernel scaffold; baseline (speedup 1.0000x reference)
import jax
import jax.numpy as jnp
from jax.experimental import pallas as pl


def kernel(output, mask, ind, target):
    raise NotImplementedError("write your pallas kernel here")


# speedup vs baseline: 1.6959x; 1.6959x over previous
import jax
import jax.numpy as jnp
from jax import lax
from jax.experimental import pallas as pl
from jax.experimental.pallas import tpu as pltpu
from jax.experimental.pallas import tpu_sc as plsc

B, D, H, W, M = 32, 4, 128, 128, 128
HW = H * W
L = 16
NS = 16
BPT = B // NS
NCHUNK = M // L


def _take16(x, idx):
    dn = lax.GatherDimensionNumbers(
        offset_dims=(), collapsed_slice_dims=(0,), start_index_map=(0,))
    return lax.gather(x, idx[:, None], dn, slice_sizes=(1,),
                      mode=lax.GatherScatterMode.PROMISE_IN_BOUNDS)


def _sc_body(out_hbm, mask_hbm, ind_hbm, targ_hbm, res_hbm, parts_hbm,
             ind_v, mask_v, targ_v, idx_v, pred_v, part_v, rows_v, outv,
             sem):
    cid = lax.axis_index("c")
    sid = lax.axis_index("s")
    lane = lax.iota(jnp.int32, L)

    @pl.when(cid == 0)
    def _work():
        pltpu.sync_copy(ind_hbm.at[pl.ds(sid * BPT, BPT)], ind_v)
        pltpu.sync_copy(mask_hbm.at[pl.ds(sid * BPT, BPT)], mask_v)
        pltpu.sync_copy(targ_hbm.at[pl.ds(sid * BPT * M * D, BPT * M * D)],
                        targ_v)

        lq = lane >> 2
        ld = lane & (D - 1)

        for bl in range(BPT):
            for c in range(NCHUNK):
                iv = ind_v[bl, pl.ds(c * L, L)]
                for q in range(4):
                    k = bl * NCHUNK * 4 + c * 4 + q
                    ivq = _take16(iv, q * 4 + lq)
                    flat = ((sid * BPT + bl) * D + ld) * HW + ivq
                    idx_v[k // 8, pl.ds((k % 8) * L, L)] = flat

        copies = [
            pltpu.async_copy(out_hbm.at[idx_v.at[r]], pred_v.at[r], sem)
            for r in range(BPT * D)
        ]
        for cp in copies:
            cp.wait()

        acc = jnp.zeros((L,), jnp.float32)
        accm = jnp.zeros((L,), jnp.float32)
        for bl in range(BPT):
            for c in range(NCHUNK):
                mvi = mask_v[bl, pl.ds(c * L, L)].astype(jnp.float32)
                accm = accm + mvi
                for q in range(4):
                    k = bl * NCHUNK * 4 + c * 4 + q
                    mv = _take16(mvi, q * 4 + lq)
                    pv = pred_v[k // 8, pl.ds((k % 8) * L, L)]
                    tv = targ_v[pl.ds(k * L, L)]
                    acc = acc + mv * jnp.abs(pv - tv)

        y = acc + _take16(acc, lane ^ 4)
        y = y + _take16(y, lane ^ 8)
        for sh in (1, 2, 4, 8):
            accm = accm + _take16(accm, lane ^ sh)
        part = jnp.where(lane < D, y, 0.0)
        part = jnp.where(lane == D, accm, part)
        part_v[...] = part
        pltpu.sync_copy(part_v, parts_hbm.at[sid])

    plsc.subcore_barrier()

    @pl.when((cid == 0) & (sid == 0))
    def _final():
        pltpu.sync_copy(parts_hbm, rows_v)
        tot = jnp.zeros((L,), jnp.float32)
        for i in range(NS):
            tot = tot + rows_v[i, :]
        msum = _take16(tot, jnp.full((L,), D, jnp.int32))
        outv[...] = jnp.where(lane < D, tot, 0.0) / (msum + 1e-4)
        pltpu.sync_copy(outv, res_hbm)


_sc_call = pl.kernel(
    _sc_body,
    out_type=(jax.ShapeDtypeStruct((L,), jnp.float32),
              jax.ShapeDtypeStruct((NS, L), jnp.float32)),
    mesh=plsc.VectorSubcoreMesh(core_axis_name="c", subcore_axis_name="s"),
    scratch_types=[
        pltpu.VMEM((BPT, M), jnp.int32),
        pltpu.VMEM((BPT, M), jnp.int32),
        pltpu.VMEM((BPT * M * D,), jnp.float32),
        pltpu.VMEM((BPT * D, M), jnp.int32),
        pltpu.VMEM((BPT * D, M), jnp.float32),
        pltpu.VMEM((L,), jnp.float32),
        pltpu.VMEM((NS, L), jnp.float32),
        pltpu.VMEM((L,), jnp.float32),
        pltpu.SemaphoreType.DMA,
    ],
)


def kernel(output, mask, ind, target):
    res, _ = _sc_call(output.reshape(-1), mask, ind, target.reshape(-1))
    return res[:D]


# speedup vs baseline: 1.8703x; 1.1028x over previous
import jax
import jax.numpy as jnp
from jax import lax
from jax.experimental import pallas as pl
from jax.experimental.pallas import tpu as pltpu
from jax.experimental.pallas import tpu_sc as plsc

B, D, H, W, M = 32, 4, 128, 128, 128
HW = H * W
L = 16
NS = 16
BPT = B // NS
NCHUNK = M // L


def _take16(x, idx):
    dn = lax.GatherDimensionNumbers(
        offset_dims=(), collapsed_slice_dims=(0,), start_index_map=(0,))
    return lax.gather(x, idx[:, None], dn, slice_sizes=(1,),
                      mode=lax.GatherScatterMode.PROMISE_IN_BOUNDS)


def _sc_body(out_hbm, mask_hbm, ind_hbm, targ_hbm, res_hbm,
             ind_v, mask_v, targ_v, idx_v, pred_v, part_v, rows_v, outv,
             parts_hbm, sem):
    sid = lax.axis_index("s")
    lane = lax.iota(jnp.int32, L)

    stage = [
        pltpu.async_copy(ind_hbm.at[pl.ds(sid * BPT, BPT)], ind_v, sem),
        pltpu.async_copy(mask_hbm.at[pl.ds(sid * BPT, BPT)], mask_v, sem),
        pltpu.async_copy(
            targ_hbm.at[pl.ds(sid * BPT * M * D, BPT * M * D)], targ_v, sem),
    ]
    for cp in stage:
        cp.wait()

    lq = lane >> 2
    ld = lane & (D - 1)

    for bl in range(BPT):
        for c in range(NCHUNK):
            iv = ind_v[bl, pl.ds(c * L, L)]
            for q in range(4):
                k = bl * NCHUNK * 4 + c * 4 + q
                ivq = _take16(iv, q * 4 + lq)
                flat = ((sid * BPT + bl) * D + ld) * HW + ivq
                idx_v[k // 8, pl.ds((k % 8) * L, L)] = flat

    copies = [
        pltpu.async_copy(out_hbm.at[idx_v.at[r]], pred_v.at[r], sem)
        for r in range(BPT * D)
    ]
    for cp in copies:
        cp.wait()

    acc = jnp.zeros((L,), jnp.float32)
    accm = jnp.zeros((L,), jnp.float32)
    for bl in range(BPT):
        for c in range(NCHUNK):
            mvi = mask_v[bl, pl.ds(c * L, L)].astype(jnp.float32)
            accm = accm + mvi
            for q in range(4):
                k = bl * NCHUNK * 4 + c * 4 + q
                mv = _take16(mvi, q * 4 + lq)
                pv = pred_v[k // 8, pl.ds((k % 8) * L, L)]
                tv = targ_v[pl.ds(k * L, L)]
                acc = acc + mv * jnp.abs(pv - tv)

    y = acc + _take16(acc, lane ^ 4)
    y = y + _take16(y, lane ^ 8)
    for sh in (1, 2, 4, 8):
        accm = accm + _take16(accm, lane ^ sh)
    part = jnp.where(lane < D, y, 0.0)
    part = jnp.where(lane == D, accm, part)
    part_v[...] = part
    pltpu.sync_copy(part_v, parts_hbm.at[sid])

    plsc.subcore_barrier()

    @pl.when(sid == 0)
    def _final():
        pltpu.sync_copy(parts_hbm, rows_v)
        tot = jnp.zeros((L,), jnp.float32)
        for i in range(NS):
            tot = tot + rows_v[i, :]
        msum = _take16(tot, jnp.full((L,), D, jnp.int32))
        outv[...] = jnp.where(lane < D, tot, 0.0) / (msum + 1e-4)
        pltpu.sync_copy(outv, res_hbm)


_sc_call = pl.kernel(
    _sc_body,
    out_type=jax.ShapeDtypeStruct((L,), jnp.float32),
    mesh=plsc.VectorSubcoreMesh(
        core_axis_name="c", subcore_axis_name="s", num_cores=1),
    scratch_types=[
        pltpu.VMEM((BPT, M), jnp.int32),
        pltpu.VMEM((BPT, M), jnp.int32),
        pltpu.VMEM((BPT * M * D,), jnp.float32),
        pltpu.VMEM((BPT * D, M), jnp.int32),
        pltpu.VMEM((BPT * D, M), jnp.float32),
        pltpu.VMEM((L,), jnp.float32),
        pltpu.VMEM((NS, L), jnp.float32),
        pltpu.VMEM((L,), jnp.float32),
        pltpu.HBM((NS, L), jnp.float32),
        pltpu.SemaphoreType.DMA,
    ],
)


def kernel(output, mask, ind, target):
    res = _sc_call(output.reshape(-1), mask, ind, target.reshape(-1))
    return res[:D]


# speedup vs baseline: 1.8808x; 1.0056x over previous
import jax
import jax.numpy as jnp
from jax import lax
from jax.experimental import pallas as pl
from jax.experimental.pallas import tpu as pltpu
from jax.experimental.pallas import tpu_sc as plsc

B, D, H, W, M = 32, 4, 128, 128, 128
HW = H * W
L = 16
NS = 16
BPT = B // NS
NJ = BPT * M // L


def _take16(x, idx):
    dn = lax.GatherDimensionNumbers(
        offset_dims=(), collapsed_slice_dims=(0,), start_index_map=(0,))
    return lax.gather(x, idx[:, None], dn, slice_sizes=(1,),
                      mode=lax.GatherScatterMode.PROMISE_IN_BOUNDS)


def _sc_body(out_hbm, mask_hbm, ind_hbm, targ_hbm, res_hbm,
             ind_v, mask_v, targ_v, idx_v, pred_v, part_v, rows_v, outv,
             parts_hbm, sem):
    sid = lax.axis_index("s")
    lane = lax.iota(jnp.int32, L)

    stage = [
        pltpu.async_copy(ind_hbm.at[pl.ds(sid * BPT * M, BPT * M)], ind_v,
                         sem),
        pltpu.async_copy(mask_hbm.at[pl.ds(sid * BPT * M, BPT * M)], mask_v,
                         sem),
        pltpu.async_copy(
            targ_hbm.at[pl.ds(sid * BPT * M * D, BPT * M * D)], targ_v, sem),
    ]
    for cp in stage:
        cp.wait()

    lq = lane >> 2
    ld = lane & (D - 1)
    sels = [q * 4 + lq for q in range(4)]
    dbase = ld * HW

    def idx_body(j, _):
        iv = ind_v[pl.ds(j * L, L)]
        bl = j // (M // L)
        base = (sid * BPT + bl) * (D * HW) + dbase
        for q in range(4):
            ivq = _take16(iv, sels[q])
            idx_v[pl.ds((j * 4 + q) * L, L)] = base + ivq
        return 0

    lax.fori_loop(0, NJ, idx_body, 0)

    copies = [
        pltpu.async_copy(out_hbm.at[idx_v.at[pl.ds(r * M, M)]],
                         pred_v.at[pl.ds(r * M, M)], sem)
        for r in range(BPT * D)
    ]
    for cp in copies:
        cp.wait()

    def sum_body(j, carry):
        acc, accm = carry
        mvi = mask_v[pl.ds(j * L, L)].astype(jnp.float32)
        accm = accm + mvi
        for q in range(4):
            mv = _take16(mvi, sels[q])
            pv = pred_v[pl.ds((j * 4 + q) * L, L)]
            tv = targ_v[pl.ds((j * 4 + q) * L, L)]
            acc = acc + mv * jnp.abs(pv - tv)
        return acc, accm

    zero = jnp.zeros((L,), jnp.float32)
    acc, accm = lax.fori_loop(0, NJ, sum_body, (zero, zero))

    y = acc + _take16(acc, lane ^ 4)
    y = y + _take16(y, lane ^ 8)
    for sh in (1, 2, 4, 8):
        accm = accm + _take16(accm, lane ^ sh)
    part = jnp.where(lane < D, y, 0.0)
    part = jnp.where(lane == D, accm, part)
    part_v[...] = part
    pltpu.sync_copy(part_v, parts_hbm.at[sid])

    plsc.subcore_barrier()

    @pl.when(sid == 0)
    def _final():
        pltpu.sync_copy(parts_hbm, rows_v)
        tot = jnp.zeros((L,), jnp.float32)
        for i in range(NS):
            tot = tot + rows_v[i, :]
        msum = _take16(tot, jnp.full((L,), D, jnp.int32))
        outv[...] = jnp.where(lane < D, tot, 0.0) / (msum + 1e-4)
        pltpu.sync_copy(outv, res_hbm)


_sc_call = pl.kernel(
    _sc_body,
    out_type=jax.ShapeDtypeStruct((L,), jnp.float32),
    mesh=plsc.VectorSubcoreMesh(
        core_axis_name="c", subcore_axis_name="s", num_cores=1),
    scratch_types=[
        pltpu.VMEM((BPT * M,), jnp.int32),
        pltpu.VMEM((BPT * M,), jnp.int32),
        pltpu.VMEM((BPT * M * D,), jnp.float32),
        pltpu.VMEM((BPT * M * D,), jnp.int32),
        pltpu.VMEM((BPT * M * D,), jnp.float32),
        pltpu.VMEM((L,), jnp.float32),
        pltpu.VMEM((NS, L), jnp.float32),
        pltpu.VMEM((L,), jnp.float32),
        pltpu.HBM((NS, L), jnp.float32),
        pltpu.SemaphoreType.DMA,
    ],
)


def kernel(output, mask, ind, target):
    res = _sc_call(output.reshape(-1), mask.reshape(-1), ind.reshape(-1),
                   target.reshape(-1))
    return res[:D]


# speedup vs baseline: 1.9152x; 1.0183x over previous
import jax
import jax.numpy as jnp
from jax import lax
from jax.experimental import pallas as pl
from jax.experimental.pallas import tpu as pltpu
from jax.experimental.pallas import tpu_sc as plsc

B, D, H, W, M = 32, 4, 128, 128, 128
HW = H * W
L = 16
NS = 16
BPT = B // NS
NJ = BPT * M // L


def _take16(x, idx):
    dn = lax.GatherDimensionNumbers(
        offset_dims=(), collapsed_slice_dims=(0,), start_index_map=(0,))
    return lax.gather(x, idx[:, None], dn, slice_sizes=(1,),
                      mode=lax.GatherScatterMode.PROMISE_IN_BOUNDS)


def _sc_body(out_hbm, mask_hbm, ind_hbm, targ_hbm, res_hbm,
             ind_v, mask_v, targ_v, idx_v, pred_v, part_v, rows_v, outv,
             parts_hbm, sem_i, sem_t, sem_g1, sem_g2):
    sid = lax.axis_index("s")
    lane = lax.iota(jnp.int32, L)

    cp_ind = pltpu.async_copy(
        ind_hbm.at[pl.ds(sid * BPT * M, BPT * M)], ind_v, sem_i)
    cp_mask = pltpu.async_copy(
        mask_hbm.at[pl.ds(sid * BPT * M, BPT * M)], mask_v, sem_t)
    cp_targ = pltpu.async_copy(
        targ_hbm.at[pl.ds(sid * BPT * M * D, BPT * M * D)], targ_v, sem_t)
    cp_ind.wait()

    lq = lane >> 2
    ld = lane & (D - 1)
    sels = [q * 4 + lq for q in range(4)]
    dbase = ld * HW

    def idx_body(j, _):
        iv = ind_v[pl.ds(j * L, L)]
        bl = j // (M // L)
        base = (sid * BPT + bl) * (D * HW) + dbase
        for q in range(4):
            ivq = _take16(iv, sels[q])
            idx_v[pl.ds((j * 4 + q) * L, L)] = base + ivq
        return 0

    NR = BPT * D
    lax.fori_loop(0, NJ // 2, idx_body, 0)
    copies1 = [
        pltpu.async_copy(out_hbm.at[idx_v.at[pl.ds(r * M, M)]],
                         pred_v.at[pl.ds(r * M, M)], sem_g1)
        for r in range(NR // 2)
    ]
    lax.fori_loop(NJ // 2, NJ, idx_body, 0)
    copies2 = [
        pltpu.async_copy(out_hbm.at[idx_v.at[pl.ds(r * M, M)]],
                         pred_v.at[pl.ds(r * M, M)], sem_g2)
        for r in range(NR // 2, NR)
    ]

    def sum_body(j, carry):
        acc, accm = carry
        mvi = mask_v[pl.ds(j * L, L)].astype(jnp.float32)
        accm = accm + mvi
        for q in range(4):
            mv = _take16(mvi, sels[q])
            pv = pred_v[pl.ds((j * 4 + q) * L, L)]
            tv = targ_v[pl.ds((j * 4 + q) * L, L)]
            acc = acc + mv * jnp.abs(pv - tv)
        return acc, accm

    cp_mask.wait()
    cp_targ.wait()
    for cp in copies1:
        cp.wait()
    zero = jnp.zeros((L,), jnp.float32)
    carry = lax.fori_loop(0, NJ // 2, sum_body, (zero, zero))
    for cp in copies2:
        cp.wait()
    acc, accm = lax.fori_loop(NJ // 2, NJ, sum_body, carry)

    y = acc + _take16(acc, lane ^ 4)
    y = y + _take16(y, lane ^ 8)
    for sh in (1, 2, 4, 8):
        accm = accm + _take16(accm, lane ^ sh)
    part = jnp.where(lane < D, y, 0.0)
    part = jnp.where(lane == D, accm, part)
    part_v[...] = part
    pltpu.sync_copy(part_v, parts_hbm.at[sid])

    plsc.subcore_barrier()

    @pl.when(sid == 0)
    def _final():
        pltpu.sync_copy(parts_hbm, rows_v)
        tot = jnp.zeros((L,), jnp.float32)
        for i in range(NS):
            tot = tot + rows_v[i, :]
        msum = _take16(tot, jnp.full((L,), D, jnp.int32))
        outv[...] = jnp.where(lane < D, tot, 0.0) / (msum + 1e-4)
        pltpu.sync_copy(outv, res_hbm)


_sc_call = pl.kernel(
    _sc_body,
    out_type=jax.ShapeDtypeStruct((L,), jnp.float32),
    mesh=plsc.VectorSubcoreMesh(
        core_axis_name="c", subcore_axis_name="s", num_cores=1),
    scratch_types=[
        pltpu.VMEM((BPT * M,), jnp.int32),
        pltpu.VMEM((BPT * M,), jnp.int32),
        pltpu.VMEM((BPT * M * D,), jnp.float32),
        pltpu.VMEM((BPT * M * D,), jnp.int32),
        pltpu.VMEM((BPT * M * D,), jnp.float32),
        pltpu.VMEM((L,), jnp.float32),
        pltpu.VMEM((NS, L), jnp.float32),
        pltpu.VMEM((L,), jnp.float32),
        pltpu.HBM((NS, L), jnp.float32),
        pltpu.SemaphoreType.DMA,
        pltpu.SemaphoreType.DMA,
        pltpu.SemaphoreType.DMA,
        pltpu.SemaphoreType.DMA,
    ],
)


def kernel(output, mask, ind, target):
    res = _sc_call(output.reshape(-1), mask.reshape(-1), ind.reshape(-1),
                   target.reshape(-1))
    return res[:D]


# speedup vs baseline: 1.9694x; 1.0283x over previous
import jax
import jax.numpy as jnp
from jax import lax
from jax.experimental import pallas as pl
from jax.experimental.pallas import tpu as pltpu
from jax.experimental.pallas import tpu_sc as plsc

B, D, H, W, M = 32, 4, 128, 128, 128
HW = H * W
L = 16
NS = 16
BPT = B // NS
NJ = BPT * M // L


def _take16(x, idx):
    dn = lax.GatherDimensionNumbers(
        offset_dims=(), collapsed_slice_dims=(0,), start_index_map=(0,))
    return lax.gather(x, idx[:, None], dn, slice_sizes=(1,),
                      mode=lax.GatherScatterMode.PROMISE_IN_BOUNDS)


def _sc_body(out_hbm, mask_hbm, ind_hbm, targ_hbm, res_hbm,
             ind_v, mask_v, targ_v, idx_v, pred_v, part_v, outv,
             shared, sem_i, sem_t, sem_g1, sem_g2):
    sid = lax.axis_index("s")
    lane = lax.iota(jnp.int32, L)

    cp_ind = pltpu.async_copy(
        ind_hbm.at[pl.ds(sid * BPT * M, BPT * M)], ind_v, sem_i)
    cp_mask = pltpu.async_copy(
        mask_hbm.at[pl.ds(sid * BPT * M, BPT * M)], mask_v, sem_t)
    cp_targ = pltpu.async_copy(
        targ_hbm.at[pl.ds(sid * BPT * M * D, BPT * M * D)], targ_v, sem_t)

    @pl.when(sid == 0)
    def _init():
        part_v[...] = jnp.zeros((L,), jnp.float32)
        pltpu.sync_copy(part_v, shared)
    plsc.subcore_barrier()

    cp_ind.wait()

    lq = lane >> 2
    ld = lane & (D - 1)
    sels = [q * 4 + lq for q in range(4)]
    dbase = ld * HW

    def idx_body(j, _):
        iv = ind_v[pl.ds(j * L, L)]
        bl = j // (M // L)
        base = (sid * BPT + bl) * (D * HW) + dbase
        for q in range(4):
            ivq = _take16(iv, sels[q])
            idx_v[pl.ds((j * 4 + q) * L, L)] = base + ivq
        return 0

    NR = BPT * D
    lax.fori_loop(0, NJ // 2, idx_body, 0)
    copies1 = [
        pltpu.async_copy(out_hbm.at[idx_v.at[pl.ds(r * M, M)]],
                         pred_v.at[pl.ds(r * M, M)], sem_g1)
        for r in range(NR // 2)
    ]
    lax.fori_loop(NJ // 2, NJ, idx_body, 0)
    copies2 = [
        pltpu.async_copy(out_hbm.at[idx_v.at[pl.ds(r * M, M)]],
                         pred_v.at[pl.ds(r * M, M)], sem_g2)
        for r in range(NR // 2, NR)
    ]

    def sum_body(j, carry):
        acc, accm = carry
        mvi = mask_v[pl.ds(j * L, L)].astype(jnp.float32)
        accm = accm + mvi
        for q in range(4):
            mv = _take16(mvi, sels[q])
            pv = pred_v[pl.ds((j * 4 + q) * L, L)]
            tv = targ_v[pl.ds((j * 4 + q) * L, L)]
            acc = acc + mv * jnp.abs(pv - tv)
        return acc, accm

    cp_mask.wait()
    cp_targ.wait()
    for cp in copies1:
        cp.wait()
    zero = jnp.zeros((L,), jnp.float32)
    carry = lax.fori_loop(0, NJ // 2, sum_body, (zero, zero))
    for cp in copies2:
        cp.wait()
    acc, accm = lax.fori_loop(NJ // 2, NJ, sum_body, carry)

    y = acc + _take16(acc, lane ^ 4)
    y = y + _take16(y, lane ^ 8)
    for sh in (1, 2, 4, 8):
        accm = accm + _take16(accm, lane ^ sh)
    part = jnp.where(lane < D, y, 0.0)
    part = jnp.where(lane == D, accm, part)
    part_v[...] = part
    pltpu.sync_copy(part_v, shared.at[lane], add=True)

    plsc.subcore_barrier()

    @pl.when(sid == 0)
    def _final():
        pltpu.sync_copy(shared, outv)
        tot = outv[...]
        msum = _take16(tot, jnp.full((L,), D, jnp.int32))
        outv[...] = jnp.where(lane < D, tot, 0.0) / (msum + 1e-4)
        pltpu.sync_copy(outv, res_hbm)


_sc_call = pl.kernel(
    _sc_body,
    out_type=jax.ShapeDtypeStruct((L,), jnp.float32),
    mesh=plsc.VectorSubcoreMesh(
        core_axis_name="c", subcore_axis_name="s", num_cores=1),
    scratch_types=[
        pltpu.VMEM((BPT * M,), jnp.int32),
        pltpu.VMEM((BPT * M,), jnp.int32),
        pltpu.VMEM((BPT * M * D,), jnp.float32),
        pltpu.VMEM((BPT * M * D,), jnp.int32),
        pltpu.VMEM((BPT * M * D,), jnp.float32),
        pltpu.VMEM((L,), jnp.float32),
        pltpu.VMEM((L,), jnp.float32),
        pltpu.VMEM_SHARED((L,), jnp.float32),
        pltpu.SemaphoreType.DMA,
        pltpu.SemaphoreType.DMA,
        pltpu.SemaphoreType.DMA,
        pltpu.SemaphoreType.DMA,
    ],
)


def kernel(output, mask, ind, target):
    res = _sc_call(output.reshape(-1), mask.reshape(-1), ind.reshape(-1),
                   target.reshape(-1))
    return res[:D]


# speedup vs baseline: 1.9705x; 1.0006x over previous
import jax
import jax.numpy as jnp
from jax import lax
from jax.experimental import pallas as pl
from jax.experimental.pallas import tpu as pltpu
from jax.experimental.pallas import tpu_sc as plsc

B, D, H, W, M = 32, 4, 128, 128, 128
HW = H * W
L = 16
NS = 16
BPT = B // NS
NJ = BPT * M // L


def _take16(x, idx):
    dn = lax.GatherDimensionNumbers(
        offset_dims=(), collapsed_slice_dims=(0,), start_index_map=(0,))
    return lax.gather(x, idx[:, None], dn, slice_sizes=(1,),
                      mode=lax.GatherScatterMode.PROMISE_IN_BOUNDS)


def _sc_body(out_hbm, mask_hbm, ind_hbm, targ_hbm, res_hbm,
             ind_v, mask_v, targ_v, idx_v, pred_v, part_v, outv,
             shared, sem_i, sem_t, sem_g1, sem_g2):
    sid = lax.axis_index("s")
    lane = lax.iota(jnp.int32, L)

    cp_ind = pltpu.async_copy(
        ind_hbm.at[pl.ds(sid * BPT * M, BPT * M)], ind_v, sem_i)
    cp_mask = pltpu.async_copy(
        mask_hbm.at[pl.ds(sid * BPT * M, BPT * M)], mask_v, sem_t)
    cp_targ = pltpu.async_copy(
        targ_hbm.at[pl.ds(sid * BPT * M * D, BPT * M * D)], targ_v, sem_t)

    @pl.when(sid == 0)
    def _init():
        part_v[...] = jnp.zeros((L,), jnp.float32)
        pltpu.sync_copy(part_v, shared)
    plsc.subcore_barrier()

    cp_ind.wait()

    lq = lane >> 2
    ld = lane & (D - 1)
    sels = [q * 4 + lq for q in range(4)]
    dbase = ld * HW

    def idx_body(j, _):
        iv = ind_v[pl.ds(j * L, L)]
        bl = j // (M // L)
        base = (sid * BPT + bl) * (D * HW) + dbase
        for q in range(4):
            ivq = _take16(iv, sels[q])
            idx_v[pl.ds((j * 4 + q) * L, L)] = base + ivq
        return 0

    NR = BPT * D

    def fire(r, sem):
        pltpu.async_copy(out_hbm.at[idx_v.at[pl.ds(r * M, M)]],
                         pred_v.at[pl.ds(r * M, M)], sem)
        return 0

    lax.fori_loop(0, NJ // 2, idx_body, 0)
    lax.fori_loop(0, NR // 2, lambda r, _: fire(r, sem_g1), 0)
    lax.fori_loop(NJ // 2, NJ, idx_body, 0)
    lax.fori_loop(NR // 2, NR, lambda r, _: fire(r, sem_g2), 0)

    def sum_body(j, carry):
        acc, accm = carry
        mvi = mask_v[pl.ds(j * L, L)].astype(jnp.float32)
        accm = accm + mvi
        for q in range(4):
            mv = _take16(mvi, sels[q])
            pv = pred_v[pl.ds((j * 4 + q) * L, L)]
            tv = targ_v[pl.ds((j * 4 + q) * L, L)]
            acc = acc + mv * jnp.abs(pv - tv)
        return acc, accm

    cp_mask.wait()
    cp_targ.wait()
    half = BPT * M * D // 2
    pltpu.make_async_copy(out_hbm.at[pl.ds(0, half)],
                          pred_v.at[pl.ds(0, half)], sem_g1).wait()
    zero = jnp.zeros((L,), jnp.float32)
    carry = lax.fori_loop(0, NJ // 2, sum_body, (zero, zero))
    pltpu.make_async_copy(out_hbm.at[pl.ds(0, half)],
                          pred_v.at[pl.ds(half, half)], sem_g2).wait()
    acc, accm = lax.fori_loop(NJ // 2, NJ, sum_body, carry)

    y = acc + _take16(acc, lane ^ 4)
    y = y + _take16(y, lane ^ 8)
    for sh in (1, 2, 4, 8):
        accm = accm + _take16(accm, lane ^ sh)
    part = jnp.where(lane < D, y, 0.0)
    part = jnp.where(lane == D, accm, part)
    part_v[...] = part
    pltpu.sync_copy(part_v, shared.at[lane], add=True)

    plsc.subcore_barrier()

    @pl.when(sid == 0)
    def _final():
        pltpu.sync_copy(shared, outv)
        tot = outv[...]
        msum = _take16(tot, jnp.full((L,), D, jnp.int32))
        outv[...] = jnp.where(lane < D, tot, 0.0) / (msum + 1e-4)
        pltpu.sync_copy(outv, res_hbm)


_sc_call = pl.kernel(
    _sc_body,
    out_type=jax.ShapeDtypeStruct((L,), jnp.float32),
    mesh=plsc.VectorSubcoreMesh(
        core_axis_name="c", subcore_axis_name="s", num_cores=1),
    scratch_types=[
        pltpu.VMEM((BPT * M,), jnp.int32),
        pltpu.VMEM((BPT * M,), jnp.int32),
        pltpu.VMEM((BPT * M * D,), jnp.float32),
        pltpu.VMEM((BPT * M * D,), jnp.int32),
        pltpu.VMEM((BPT * M * D,), jnp.float32),
        pltpu.VMEM((L,), jnp.float32),
        pltpu.VMEM((L,), jnp.float32),
        pltpu.VMEM_SHARED((L,), jnp.float32),
        pltpu.SemaphoreType.DMA,
        pltpu.SemaphoreType.DMA,
        pltpu.SemaphoreType.DMA,
        pltpu.SemaphoreType.DMA,
    ],
)


def kernel(output, mask, ind, target):
    res = _sc_call(output.reshape(-1), mask.reshape(-1), ind.reshape(-1),
                   target.reshape(-1))
    return res[:D]


# speedup vs baseline: 1.9715x; 1.0005x over previous
import jax
import jax.numpy as jnp
from jax import lax
from jax.experimental import pallas as pl
from jax.experimental.pallas import tpu as pltpu
from jax.experimental.pallas import tpu_sc as plsc

B, D, H, W, M = 32, 4, 128, 128, 128
HW = H * W
L = 16
NS = 16
BPT = B // NS
NJ = BPT * M // L


def _take16(x, idx):
    dn = lax.GatherDimensionNumbers(
        offset_dims=(), collapsed_slice_dims=(0,), start_index_map=(0,))
    return lax.gather(x, idx[:, None], dn, slice_sizes=(1,),
                      mode=lax.GatherScatterMode.PROMISE_IN_BOUNDS)


def _sc_body(out_hbm, mask_hbm, ind_hbm, targ_hbm, res_hbm,
             ind_v, mask_v, targ_v, idx_v, pred_v, part_v, outv,
             shared, sem_i, sem_t, *sem_g):
    sid = lax.axis_index("s")
    lane = lax.iota(jnp.int32, L)

    cp_ind = pltpu.async_copy(
        ind_hbm.at[pl.ds(sid * BPT * M, BPT * M)], ind_v, sem_i)
    cp_mask = pltpu.async_copy(
        mask_hbm.at[pl.ds(sid * BPT * M, BPT * M)], mask_v, sem_t)
    cp_targ = pltpu.async_copy(
        targ_hbm.at[pl.ds(sid * BPT * M * D, BPT * M * D)], targ_v, sem_t)

    @pl.when(sid == 0)
    def _init():
        part_v[...] = jnp.zeros((L,), jnp.float32)
        pltpu.sync_copy(part_v, shared)
    plsc.subcore_barrier()

    cp_ind.wait()

    lq = lane >> 2
    ld = lane & (D - 1)
    sels = [q * 4 + lq for q in range(4)]
    dbase = ld * HW

    def idx_body(j, _):
        iv = ind_v[pl.ds(j * L, L)]
        bl = j // (M // L)
        base = (sid * BPT + bl) * (D * HW) + dbase
        for q in range(4):
            ivq = _take16(iv, sels[q])
            idx_v[pl.ds((j * 4 + q) * L, L)] = base + ivq
        return 0

    NR = BPT * D

    def fire(r, sem):
        pltpu.async_copy(out_hbm.at[idx_v.at[pl.ds(r * M, M)]],
                         pred_v.at[pl.ds(r * M, M)], sem)
        return 0

    NG = len(sem_g)
    for g in range(NG):
        lax.fori_loop(g * NJ // NG, (g + 1) * NJ // NG, idx_body, 0)
        lax.fori_loop(g * NR // NG, (g + 1) * NR // NG,
                      lambda r, _, s=sem_g[g]: fire(r, s), 0)

    def sum_body(j, carry):
        acc, accm = carry
        mvi = mask_v[pl.ds(j * L, L)].astype(jnp.float32)
        accm = accm + mvi
        for q in range(4):
            mv = _take16(mvi, sels[q])
            pv = pred_v[pl.ds((j * 4 + q) * L, L)]
            tv = targ_v[pl.ds((j * 4 + q) * L, L)]
            acc = acc + mv * jnp.abs(pv - tv)
        return acc, accm

    cp_mask.wait()
    cp_targ.wait()
    quarter = BPT * M * D // NG
    zero = jnp.zeros((L,), jnp.float32)
    carry = (zero, zero)
    for g in range(NG):
        pltpu.make_async_copy(
            out_hbm.at[pl.ds(0, quarter)],
            pred_v.at[pl.ds(g * quarter, quarter)], sem_g[g]).wait()
        carry = lax.fori_loop(g * NJ // NG, (g + 1) * NJ // NG,
                              sum_body, carry)
    acc, accm = carry

    y = acc + _take16(acc, lane ^ 4)
    y = y + _take16(y, lane ^ 8)
    for sh in (1, 2, 4, 8):
        accm = accm + _take16(accm, lane ^ sh)
    part = jnp.where(lane < D, y, 0.0)
    part = jnp.where(lane == D, accm, part)
    part_v[...] = part
    pltpu.sync_copy(part_v, shared.at[lane], add=True)

    plsc.subcore_barrier()

    @pl.when(sid == 0)
    def _final():
        pltpu.sync_copy(shared, outv)
        tot = outv[...]
        msum = _take16(tot, jnp.full((L,), D, jnp.int32))
        outv[...] = jnp.where(lane < D, tot, 0.0) / (msum + 1e-4)
        pltpu.sync_copy(outv, res_hbm)


_sc_call = pl.kernel(
    _sc_body,
    out_type=jax.ShapeDtypeStruct((L,), jnp.float32),
    mesh=plsc.VectorSubcoreMesh(
        core_axis_name="c", subcore_axis_name="s", num_cores=1),
    scratch_types=[
        pltpu.VMEM((BPT * M,), jnp.int32),
        pltpu.VMEM((BPT * M,), jnp.int32),
        pltpu.VMEM((BPT * M * D,), jnp.float32),
        pltpu.VMEM((BPT * M * D,), jnp.int32),
        pltpu.VMEM((BPT * M * D,), jnp.float32),
        pltpu.VMEM((L,), jnp.float32),
        pltpu.VMEM((L,), jnp.float32),
        pltpu.VMEM_SHARED((L,), jnp.float32),
        pltpu.SemaphoreType.DMA,
        pltpu.SemaphoreType.DMA,
        pltpu.SemaphoreType.DMA,
        pltpu.SemaphoreType.DMA,
        pltpu.SemaphoreType.DMA,
        pltpu.SemaphoreType.DMA,
    ],
)


def kernel(output, mask, ind, target):
    res = _sc_call(output.reshape(-1), mask.reshape(-1), ind.reshape(-1),
                   target.reshape(-1))
    return res[:D]
